# Initial kernel scaffold; baseline (speedup 1.0000x reference)
import jax
import jax.numpy as jnp
from jax.experimental import pallas as pl


def kernel(x, edge_index, batch, W_self, W_neigh, W0, W1):
    raise NotImplementedError("write your pallas kernel here")


# speedup vs baseline: 2.5929x; 2.5929x over previous
import functools

import jax
import jax.numpy as jnp
from jax import lax
from jax.experimental import pallas as pl
from jax.experimental.pallas import tpu as pltpu
from jax.experimental.pallas import tpu_sc as plsc

N = 10000
E = 320000
D = 128
G = 64

NC = 2
NS = 16
HALF = 5120
GRP = 128
ACC_N = HALF + GRP
PAD_N = NC * HALF
E_PAD = 327680
EPS = E_PAD // NS
NGRP = EPS // GRP
NROW = E_PAD // GRP
DEGW = 16
ZROWS = ACC_N // NS
OROWS = HALF // NS
L = 16


def _remap(dst_v, adj_v, g, base, iota):
    for l in range(GRP // L):
        dv = dst_v[g, pl.ds(l * L, L)]
        rel = dv - base
        inr = (rel >= 0) & (rel < HALF)
        trash = HALF + l * L + iota
        adj_v[pl.ds(l * L, L)] = jnp.where(inr, rel, trash)


def _zero_acc(zrow_v, acc_sh, row0):
    nfull, rem = divmod(ZROWS, GRP)
    for k in range(nfull):
        pltpu.sync_copy(zrow_v, acc_sh.at[pl.ds(row0 + k * GRP, GRP)])
    if rem:
        pltpu.sync_copy(zrow_v.at[pl.ds(0, rem)],
                        acc_sh.at[pl.ds(row0 + nfull * GRP, rem)])


def _sc_agg_body(src_hbm, dst_hbm, x_hbm, zrow_hbm,
                 agg_out,
                 src_v, dst_v, rows_v, adj_v, zrow_v,
                 agg_sh, sem):
    c = lax.axis_index("c")
    s = lax.axis_index("s")
    base = c * HALF

    pltpu.sync_copy(zrow_hbm, zrow_v)
    _zero_acc(zrow_v, agg_sh, s * ZROWS)

    g0 = s * NGRP
    pltpu.sync_copy(src_hbm.at[pl.ds(g0, NGRP)], src_v)
    pltpu.sync_copy(dst_hbm.at[pl.ds(g0, NGRP)], dst_v)

    plsc.subcore_barrier()

    iota = lax.broadcasted_iota(jnp.int32, (L,), 0)

    def body(g, carry):
        pltpu.async_copy(x_hbm.at[src_v.at[g]], rows_v, sem).wait()
        _remap(dst_v, adj_v, g, base, iota)
        pltpu.sync_copy(rows_v, agg_sh.at[adj_v], add=True)
        return carry

    lax.fori_loop(0, NGRP, body, 0)

    plsc.subcore_barrier()

    o0 = s * OROWS
    pltpu.sync_copy(agg_sh.at[pl.ds(o0, OROWS)],
                    agg_out.at[c, pl.ds(o0, OROWS)])


_sc_agg = functools.partial(
    pl.kernel,
    out_type=pltpu.HBM((NC, HALF, D), jnp.float32),
    mesh=plsc.VectorSubcoreMesh(core_axis_name="c", subcore_axis_name="s"),
    scratch_types=[
        pltpu.VMEM((NGRP, GRP), jnp.int32),
        pltpu.VMEM((NGRP, GRP), jnp.int32),
        pltpu.VMEM((GRP, D), jnp.float32),
        pltpu.VMEM((GRP,), jnp.int32),
        pltpu.VMEM((GRP, D), jnp.float32),
        pltpu.VMEM_SHARED((ACC_N, D), jnp.float32),
        pltpu.SemaphoreType.DMA,
    ],
)(_sc_agg_body)


def _sc_deg_body(dst_hbm, ones_hbm, zde_hbm,
                 deg_out,
                 dst_v, adj_v, ones_v, zde_v, degbuf_v,
                 deg_sh):
    c = lax.axis_index("c")
    s = lax.axis_index("s")
    base = c * HALF

    pltpu.sync_copy(ones_hbm, ones_v)
    pltpu.sync_copy(zde_hbm, zde_v)
    pltpu.sync_copy(zde_v, deg_sh.at[pl.ds(s * ZROWS, ZROWS)])

    g0 = s * NGRP
    pltpu.sync_copy(dst_hbm.at[pl.ds(g0, NGRP)], dst_v)

    plsc.subcore_barrier()

    iota = lax.broadcasted_iota(jnp.int32, (L,), 0)

    def body(g, carry):
        _remap(dst_v, adj_v, g, base, iota)
        pltpu.sync_copy(ones_v, deg_sh.at[adj_v], add=True)
        return carry

    lax.fori_loop(0, NGRP, body, 0)

    plsc.subcore_barrier()

    o0 = s * OROWS
    pltpu.sync_copy(deg_sh.at[pl.ds(o0, OROWS)], degbuf_v)
    pltpu.sync_copy(degbuf_v, deg_out.at[pl.ds(c * HALF + o0, OROWS)])


_sc_deg = functools.partial(
    pl.kernel,
    out_type=pltpu.HBM((PAD_N,), jnp.float32),
    mesh=plsc.VectorSubcoreMesh(core_axis_name="c", subcore_axis_name="s"),
    scratch_types=[
        pltpu.VMEM((NGRP, GRP), jnp.int32),
        pltpu.VMEM((GRP,), jnp.int32),
        pltpu.VMEM((GRP,), jnp.float32),
        pltpu.VMEM((ZROWS,), jnp.float32),
        pltpu.VMEM((OROWS,), jnp.float32),
        pltpu.VMEM_SHARED((ACC_N,), jnp.float32),
    ],
)(_sc_deg_body)


BN = 640
NBLK = PAD_N // BN
BPC = HALF // BN


def _tc_body(x_ref, agg_ref, deg_ref, batch_ref,
             ws_ref, wn_ref, w0_ref, w1_ref, out_ref,
             pooled_acc, cnt_acc):
    i = pl.program_id(0)

    @pl.when(i == 0)
    def _():
        pooled_acc[...] = jnp.zeros_like(pooled_acc)
        cnt_acc[...] = jnp.zeros_like(cnt_acc)

    deg = jnp.maximum(deg_ref[...], 1.0)
    agg = agg_ref[0] / deg
    h = (jnp.dot(x_ref[...], ws_ref[...], preferred_element_type=jnp.float32)
         + jnp.dot(agg, wn_ref[...], preferred_element_type=jnp.float32))
    h = jnp.maximum(h, 0.0)

    onehot = (batch_ref[...] ==
              lax.broadcasted_iota(jnp.int32, (BN, G), 1)).astype(jnp.float32)
    pooled_acc[...] += lax.dot_general(
        onehot, h, (((0,), (0,)), ((), ())),
        preferred_element_type=jnp.float32)
    cnt_acc[...] += jnp.sum(onehot, axis=0)[:, None]

    @pl.when(i == NBLK - 1)
    def _():
        pooled = pooled_acc[...] / jnp.maximum(cnt_acc[...], 1.0)
        mid = jnp.maximum(
            jnp.dot(pooled, w0_ref[...], preferred_element_type=jnp.float32),
            0.0)
        out_ref[...] = jnp.dot(mid, w1_ref[...],
                               preferred_element_type=jnp.float32)


_tc_call = pl.pallas_call(
    _tc_body,
    grid=(NBLK,),
    in_specs=[
        pl.BlockSpec((BN, D), lambda i: (i, 0)),
        pl.BlockSpec((1, BN, D), lambda i: (i // BPC, i % BPC, 0)),
        pl.BlockSpec((BN, 1), lambda i: (i, 0)),
        pl.BlockSpec((BN, 1), lambda i: (i, 0)),
        pl.BlockSpec((D, D), lambda i: (0, 0)),
        pl.BlockSpec((D, D), lambda i: (0, 0)),
        pl.BlockSpec((D, D), lambda i: (0, 0)),
        pl.BlockSpec((D, D), lambda i: (0, 0)),
    ],
    out_specs=pl.BlockSpec((G, D), lambda i: (0, 0)),
    out_shape=jax.ShapeDtypeStruct((G, D), jnp.float32),
    scratch_shapes=[
        pltpu.VMEM((G, D), jnp.float32),
        pltpu.VMEM((G, 1), jnp.float32),
    ],
)


def kernel(x, edge_index, batch, W_self, W_neigh, W0, W1):
    src = edge_index[0]
    dst = edge_index[1]
    pad = jnp.full((E_PAD - E,), N, jnp.int32)
    src_p = jnp.concatenate([src, pad]).reshape(NROW, GRP)
    dst_p = jnp.concatenate([dst, pad]).reshape(NROW, GRP)
    x_p = jnp.pad(x, ((0, PAD_N - N), (0, 0)))
    batch_p = jnp.pad(batch, (0, PAD_N - N), constant_values=G).reshape(
        PAD_N, 1)

    ones1 = jnp.ones((GRP,), jnp.float32)
    zrow = jnp.zeros((GRP, D), jnp.float32)
    zde = jnp.zeros((ZROWS,), jnp.float32)

    agg_parts = _sc_agg(src_p, dst_p, x_p, zrow)
    deg_parts = _sc_deg(dst_p, ones1, zde)

    deg3 = deg_parts.reshape(PAD_N, 1)
    return _tc_call(x_p, agg_parts, deg3, batch_p,
                    W_self, W_neigh, W0, W1)


# speedup vs baseline: 2.6475x; 1.0211x over previous
import functools

import jax
import jax.numpy as jnp
from jax import lax
from jax.experimental import pallas as pl
from jax.experimental.pallas import tpu as pltpu
from jax.experimental.pallas import tpu_sc as plsc

N = 10000
E = 320000
D = 128
G = 64

NC = 2
NS = 16
HALF = 5120
GRP = 128
ACC_N = HALF + GRP
PAD_N = NC * HALF
E_PAD = 327680
EPS = E_PAD // NS
NGRP = EPS // GRP
NROW = E_PAD // GRP
DEGW = 16
ZROWS = ACC_N // NS
OROWS = HALF // NS
L = 16


def _remap(dst_v, adj_v, g, base, iota):
    for l in range(GRP // L):
        dv = dst_v[g, pl.ds(l * L, L)]
        rel = dv - base
        inr = (rel >= 0) & (rel < HALF)
        trash = HALF + l * L + iota
        adj_v[pl.ds(l * L, L)] = jnp.where(inr, rel, trash)


def _zero_acc(zrow_v, acc_sh, row0):
    nfull, rem = divmod(ZROWS, GRP)
    for k in range(nfull):
        pltpu.sync_copy(zrow_v, acc_sh.at[pl.ds(row0 + k * GRP, GRP)])
    if rem:
        pltpu.sync_copy(zrow_v.at[pl.ds(0, rem)],
                        acc_sh.at[pl.ds(row0 + nfull * GRP, rem)])


def _sc_agg_body(src_hbm, dst_hbm, x_hbm, zrow_hbm, ones_hbm, zde_hbm,
                 agg_out, deg_out,
                 src_v, dst_v, rows_v, adj_v, zrow_v, ones_v, zde_v, degbuf_v,
                 agg_sh, deg_sh, sem):
    c = lax.axis_index("c")
    s = lax.axis_index("s")
    base = c * HALF

    pltpu.sync_copy(zrow_hbm, zrow_v)
    pltpu.sync_copy(ones_hbm, ones_v)
    pltpu.sync_copy(zde_hbm, zde_v)
    _zero_acc(zrow_v, agg_sh, s * ZROWS)
    pltpu.sync_copy(zde_v, deg_sh.at[pl.ds(s * ZROWS, ZROWS)])

    g0 = s * NGRP
    pltpu.sync_copy(src_hbm.at[pl.ds(g0, NGRP)], src_v)
    pltpu.sync_copy(dst_hbm.at[pl.ds(g0, NGRP)], dst_v)

    plsc.subcore_barrier()

    iota = lax.broadcasted_iota(jnp.int32, (L,), 0)

    def body(g, carry):
        pltpu.async_copy(x_hbm.at[src_v.at[g]], rows_v, sem).wait()
        _remap(dst_v, adj_v, g, base, iota)
        pltpu.sync_copy(rows_v, agg_sh.at[adj_v], add=True)
        pltpu.sync_copy(ones_v, deg_sh.at[adj_v], add=True)
        return carry

    lax.fori_loop(0, NGRP, body, 0)

    plsc.subcore_barrier()

    o0 = s * OROWS
    pltpu.sync_copy(agg_sh.at[pl.ds(o0, OROWS)],
                    agg_out.at[c, pl.ds(o0, OROWS)])
    pltpu.sync_copy(deg_sh.at[pl.ds(o0, OROWS)], degbuf_v)
    pltpu.sync_copy(degbuf_v, deg_out.at[pl.ds(c * HALF + o0, OROWS)])


_sc_agg = functools.partial(
    pl.kernel,
    out_type=(pltpu.HBM((NC, HALF, D), jnp.float32),
              pltpu.HBM((PAD_N,), jnp.float32)),
    mesh=plsc.VectorSubcoreMesh(core_axis_name="c", subcore_axis_name="s"),
    scratch_types=[
        pltpu.VMEM((NGRP, GRP), jnp.int32),
        pltpu.VMEM((NGRP, GRP), jnp.int32),
        pltpu.VMEM((GRP, D), jnp.float32),
        pltpu.VMEM((GRP,), jnp.int32),
        pltpu.VMEM((GRP, D), jnp.float32),
        pltpu.VMEM((GRP,), jnp.float32),
        pltpu.VMEM((ZROWS,), jnp.float32),
        pltpu.VMEM((OROWS,), jnp.float32),
        pltpu.VMEM_SHARED((ACC_N, D), jnp.float32),
        pltpu.VMEM_SHARED((ACC_N,), jnp.float32),
        pltpu.SemaphoreType.DMA,
    ],
)(_sc_agg_body)


BN = 640
NBLK = PAD_N // BN
BPC = HALF // BN


def _tc_body(x_ref, agg_ref, deg_ref, batch_ref,
             ws_ref, wn_ref, w0_ref, w1_ref, out_ref,
             pooled_acc, cnt_acc):
    i = pl.program_id(0)

    @pl.when(i == 0)
    def _():
        pooled_acc[...] = jnp.zeros_like(pooled_acc)
        cnt_acc[...] = jnp.zeros_like(cnt_acc)

    deg = jnp.maximum(deg_ref[...], 1.0)
    agg = agg_ref[0] / deg
    h = (jnp.dot(x_ref[...], ws_ref[...], preferred_element_type=jnp.float32)
         + jnp.dot(agg, wn_ref[...], preferred_element_type=jnp.float32))
    h = jnp.maximum(h, 0.0)

    onehot = (batch_ref[...] ==
              lax.broadcasted_iota(jnp.int32, (BN, G), 1)).astype(jnp.float32)
    pooled_acc[...] += lax.dot_general(
        onehot, h, (((0,), (0,)), ((), ())),
        preferred_element_type=jnp.float32)
    cnt_acc[...] += jnp.sum(onehot, axis=0)[:, None]

    @pl.when(i == NBLK - 1)
    def _():
        pooled = pooled_acc[...] / jnp.maximum(cnt_acc[...], 1.0)
        mid = jnp.maximum(
            jnp.dot(pooled, w0_ref[...], preferred_element_type=jnp.float32),
            0.0)
        out_ref[...] = jnp.dot(mid, w1_ref[...],
                               preferred_element_type=jnp.float32)


_tc_call = pl.pallas_call(
    _tc_body,
    grid=(NBLK,),
    in_specs=[
        pl.BlockSpec((BN, D), lambda i: (i, 0)),
        pl.BlockSpec((1, BN, D), lambda i: (i // BPC, i % BPC, 0)),
        pl.BlockSpec((BN, 1), lambda i: (i, 0)),
        pl.BlockSpec((BN, 1), lambda i: (i, 0)),
        pl.BlockSpec((D, D), lambda i: (0, 0)),
        pl.BlockSpec((D, D), lambda i: (0, 0)),
        pl.BlockSpec((D, D), lambda i: (0, 0)),
        pl.BlockSpec((D, D), lambda i: (0, 0)),
    ],
    out_specs=pl.BlockSpec((G, D), lambda i: (0, 0)),
    out_shape=jax.ShapeDtypeStruct((G, D), jnp.float32),
    scratch_shapes=[
        pltpu.VMEM((G, D), jnp.float32),
        pltpu.VMEM((G, 1), jnp.float32),
    ],
)


def kernel(x, edge_index, batch, W_self, W_neigh, W0, W1):
    src = edge_index[0]
    dst = edge_index[1]
    pad = jnp.full((E_PAD - E,), N, jnp.int32)
    src_p = jnp.concatenate([src, pad]).reshape(NROW, GRP)
    dst_p = jnp.concatenate([dst, pad]).reshape(NROW, GRP)
    x_p = jnp.pad(x, ((0, PAD_N - N), (0, 0)))
    batch_p = jnp.pad(batch, (0, PAD_N - N), constant_values=G).reshape(
        PAD_N, 1)

    ones1 = jnp.ones((GRP,), jnp.float32)
    zrow = jnp.zeros((GRP, D), jnp.float32)
    zde = jnp.zeros((ZROWS,), jnp.float32)

    agg_parts, deg_parts = _sc_agg(src_p, dst_p, x_p, zrow, ones1, zde)

    deg3 = deg_parts.reshape(PAD_N, 1)
    return _tc_call(x_p, agg_parts, deg3, batch_p,
                    W_self, W_neigh, W0, W1)


# speedup vs baseline: 2.9640x; 1.1195x over previous
import functools

import jax
import jax.numpy as jnp
from jax import lax
from jax.experimental import pallas as pl
from jax.experimental.pallas import tpu as pltpu
from jax.experimental.pallas import tpu_sc as plsc

N = 10000
E = 320000
D = 128
G = 64

NC = 2
NS = 16
HALF = 5120
GRP = 128
ACC_N = HALF + GRP
PAD_N = NC * HALF
E_PAD = 327680
EPS = E_PAD // NS
NGRP = EPS // GRP
NROW = E_PAD // GRP
ZROWS = ACC_N // NS
OROWS = HALF // NS
L = 16


def _sc_agg_body(src_hbm, dst_hbm, x_hbm, zrow_hbm, ones_hbm, zde_hbm,
                 agg_out, deg_out,
                 src_v, dst_v, rows0_v, rows1_v,
                 ones_v, zde_v, degbuf_v,
                 agg_sh, deg_sh, sem, dsem):
    c = lax.axis_index("c")
    s = lax.axis_index("s")
    base = c * HALF

    pltpu.sync_copy(zrow_hbm, rows0_v)
    pltpu.sync_copy(zde_hbm, zde_v)
    row0 = s * ZROWS
    nfull, rem = divmod(ZROWS, GRP)
    for k in range(nfull):
        pltpu.sync_copy(rows0_v, agg_sh.at[pl.ds(row0 + k * GRP, GRP)])
    if rem:
        pltpu.sync_copy(rows0_v.at[pl.ds(0, rem)],
                        agg_sh.at[pl.ds(row0 + nfull * GRP, rem)])
    pltpu.sync_copy(zde_v, deg_sh.at[pl.ds(row0, ZROWS)])

    g0 = s * NGRP
    pltpu.sync_copy(src_hbm.at[pl.ds(g0, NGRP)], src_v)
    pltpu.sync_copy(dst_hbm.at[pl.ds(g0, NGRP)], dst_v)
    pltpu.sync_copy(ones_hbm, ones_v)

    iota = lax.broadcasted_iota(jnp.int32, (L,), 0)

    def pre(g, carry):
        for l in range(GRP // L):
            dv = dst_v[g, pl.ds(l * L, L)]
            rel = dv - base
            inr = (rel >= 0) & (rel < HALF)
            trash = HALF + l * L + iota
            dst_v[g, pl.ds(l * L, L)] = jnp.where(inr, rel, trash)
        return carry

    lax.fori_loop(0, NGRP, pre, 0)

    plsc.subcore_barrier()

    pltpu.async_copy(x_hbm.at[src_v.at[0]], rows0_v, sem)
    pltpu.async_copy(x_hbm.at[src_v.at[1]], rows1_v, sem)

    def body(i, carry):
        gbase = i * 2
        for b, buf in enumerate((rows0_v, rows1_v)):
            g = gbase + b
            pltpu.make_async_copy(x_hbm.at[src_v.at[g]], buf, sem).wait()
            pltpu.sync_copy(buf, agg_sh.at[dst_v.at[g]], add=True)
            pltpu.async_copy(ones_v, deg_sh.at[dst_v.at[g]], dsem, add=True)
            nxt = g + 2

            @pl.when(nxt < NGRP)
            def _():
                pltpu.async_copy(x_hbm.at[src_v.at[nxt]], buf, sem)
        return carry

    lax.fori_loop(0, NGRP // 2, body, 0)

    def drain(g, carry):
        pltpu.make_async_copy(ones_v, deg_sh.at[dst_v.at[g]], dsem).wait()
        return carry

    lax.fori_loop(0, NGRP, drain, 0)
    plsc.subcore_barrier()

    o0 = s * OROWS
    pltpu.sync_copy(agg_sh.at[pl.ds(o0, OROWS)],
                    agg_out.at[c, pl.ds(o0, OROWS)])
    pltpu.sync_copy(deg_sh.at[pl.ds(o0, OROWS)], degbuf_v)
    pltpu.sync_copy(degbuf_v, deg_out.at[pl.ds(c * HALF + o0, OROWS)])


_sc_agg = functools.partial(
    pl.kernel,
    out_type=(pltpu.HBM((NC, HALF, D), jnp.float32),
              pltpu.HBM((PAD_N,), jnp.float32)),
    mesh=plsc.VectorSubcoreMesh(core_axis_name="c", subcore_axis_name="s"),
    scratch_types=[
        pltpu.VMEM((NGRP, GRP), jnp.int32),
        pltpu.VMEM((NGRP, GRP), jnp.int32),
        pltpu.VMEM((GRP, D), jnp.float32),
        pltpu.VMEM((GRP, D), jnp.float32),
        pltpu.VMEM((GRP,), jnp.float32),
        pltpu.VMEM((ZROWS,), jnp.float32),
        pltpu.VMEM((OROWS,), jnp.float32),
        pltpu.VMEM_SHARED((ACC_N, D), jnp.float32),
        pltpu.VMEM_SHARED((ACC_N,), jnp.float32),
        pltpu.SemaphoreType.DMA,
        pltpu.SemaphoreType.DMA,
    ],
)(_sc_agg_body)


BN = 640
NBLK = PAD_N // BN
BPC = HALF // BN


def _tc_body(x_ref, agg_ref, deg_ref, batch_ref,
             ws_ref, wn_ref, w0_ref, w1_ref, out_ref,
             pooled_acc, cnt_acc):
    i = pl.program_id(0)

    @pl.when(i == 0)
    def _():
        pooled_acc[...] = jnp.zeros_like(pooled_acc)
        cnt_acc[...] = jnp.zeros_like(cnt_acc)

    deg = jnp.maximum(deg_ref[...], 1.0)
    agg = agg_ref[0] / deg
    h = (jnp.dot(x_ref[...], ws_ref[...], preferred_element_type=jnp.float32)
         + jnp.dot(agg, wn_ref[...], preferred_element_type=jnp.float32))
    h = jnp.maximum(h, 0.0)

    onehot = (batch_ref[...] ==
              lax.broadcasted_iota(jnp.int32, (BN, G), 1)).astype(jnp.float32)
    pooled_acc[...] += lax.dot_general(
        onehot, h, (((0,), (0,)), ((), ())),
        preferred_element_type=jnp.float32)
    cnt_acc[...] += jnp.sum(onehot, axis=0)[:, None]

    @pl.when(i == NBLK - 1)
    def _():
        pooled = pooled_acc[...] / jnp.maximum(cnt_acc[...], 1.0)
        mid = jnp.maximum(
            jnp.dot(pooled, w0_ref[...], preferred_element_type=jnp.float32),
            0.0)
        out_ref[...] = jnp.dot(mid, w1_ref[...],
                               preferred_element_type=jnp.float32)


_tc_call = pl.pallas_call(
    _tc_body,
    grid=(NBLK,),
    in_specs=[
        pl.BlockSpec((BN, D), lambda i: (i, 0)),
        pl.BlockSpec((1, BN, D), lambda i: (i // BPC, i % BPC, 0)),
        pl.BlockSpec((BN, 1), lambda i: (i, 0)),
        pl.BlockSpec((BN, 1), lambda i: (i, 0)),
        pl.BlockSpec((D, D), lambda i: (0, 0)),
        pl.BlockSpec((D, D), lambda i: (0, 0)),
        pl.BlockSpec((D, D), lambda i: (0, 0)),
        pl.BlockSpec((D, D), lambda i: (0, 0)),
    ],
    out_specs=pl.BlockSpec((G, D), lambda i: (0, 0)),
    out_shape=jax.ShapeDtypeStruct((G, D), jnp.float32),
    scratch_shapes=[
        pltpu.VMEM((G, D), jnp.float32),
        pltpu.VMEM((G, 1), jnp.float32),
    ],
)


def kernel(x, edge_index, batch, W_self, W_neigh, W0, W1):
    src = edge_index[0]
    dst = edge_index[1]
    pad = jnp.full((E_PAD - E,), N, jnp.int32)
    src_p = jnp.concatenate([src, pad]).reshape(NROW, GRP)
    dst_p = jnp.concatenate([dst, pad]).reshape(NROW, GRP)
    x_p = jnp.pad(x, ((0, PAD_N - N), (0, 0)))
    batch_p = jnp.pad(batch, (0, PAD_N - N), constant_values=G).reshape(
        PAD_N, 1)

    ones1 = jnp.ones((GRP,), jnp.float32)
    zrow = jnp.zeros((GRP, D), jnp.float32)
    zde = jnp.zeros((ZROWS,), jnp.float32)

    agg_parts, deg_parts = _sc_agg(src_p, dst_p, x_p, zrow, ones1, zde)

    deg3 = deg_parts.reshape(PAD_N, 1)
    return _tc_call(x_p, agg_parts, deg3, batch_p,
                    W_self, W_neigh, W0, W1)
